# baseline (speedup 1.0000x reference)
import jax
import jax.numpy as jnp
from jax import lax
from jax.experimental import pallas as pl
from jax.experimental.pallas import tpu as pltpu
from jax.experimental.pallas import tpu_sc as plsc

NC = 2
NS = 16
NW = NC * NS
NP = NW // 2

ROWS_P = 256
GRP = 64
DH = 256
UNITS = 50 * (ROWS_P // GRP)


def _body(table_hbm, idx_hbm, out_hbm, tab_v, idx_v, st0, st1, w0, w1):
    wid = lax.axis_index("s") * NC + lax.axis_index("c")
    pair = wid // 2
    h = lax.rem(wid, 2)
    col0 = pair * ROWS_P
    stage = (st0, st1)
    wsem = (w0, w1)

    pltpu.sync_copy(table_hbm.at[:, pl.ds(h * DH, DH)], tab_v)
    pltpu.sync_copy(idx_hbm.at[pair], idx_v)

    lane = lax.iota(jnp.int32, 16)

    def dst_of(u):
        j = u // 4
        g = lax.rem(u, 4)
        return out_hbm.at[j, pl.ds(col0 + g * GRP, GRP), pl.ds(h * DH, DH)]

    def expand(u, b):
        def row16(r16, carry):
            r0 = r16 * 16
            iv = idx_v[pl.ds(u * GRP + r0, 16)]
            for j in range(16):
                rsplat = lax.gather(
                    iv,
                    jnp.full((16, 1), j, jnp.int32),
                    lax.GatherDimensionNumbers(
                        offset_dims=(), collapsed_slice_dims=(0,), start_index_map=(0,)
                    ),
                    (1,),
                    mode=lax.GatherScatterMode.PROMISE_IN_BOUNDS,
                )
                for k in range(DH // 16):
                    v = plsc.load_gather(tab_v, [rsplat, lane + (k * 16)])
                    stage[b][r0 + j, pl.ds(k * 16, 16)] = v
            return carry

        lax.fori_loop(0, GRP // 16, row16, 0)

    def step(gi, carry):
        for b in range(2):
            u = gi * 2 + b

            @pl.when(u >= 2)
            def _():
                pltpu.make_async_copy(stage[b], dst_of(u - 2), wsem[b]).wait()

            expand(u, b)
            pltpu.async_copy(stage[b], dst_of(u), wsem[b])

        return carry

    lax.fori_loop(0, UNITS // 2, step, 0)

    pltpu.make_async_copy(st0, dst_of(UNITS - 2), w0).wait()
    pltpu.make_async_copy(st1, dst_of(UNITS - 1), w1).wait()


def kernel(ids, kernel):
    table = kernel
    n_rows, d = table.shape
    nb_rows, seq = ids.shape
    assert nb_rows == NP * ROWS_P and d == 2 * DH

    idx = (
        ids.astype(jnp.int32)
        .T.reshape(seq, NP, ROWS_P // GRP, GRP)
        .transpose(1, 0, 2, 3)
        .reshape(NP, UNITS * GRP)
    )

    mesh = plsc.VectorSubcoreMesh(
        core_axis_name="c", subcore_axis_name="s", num_cores=NC, num_subcores=NS
    )
    run = pl.kernel(
        _body,
        out_type=jax.ShapeDtypeStruct((seq, nb_rows, d), table.dtype),
        mesh=mesh,
        compiler_params=pltpu.CompilerParams(needs_layout_passes=False),
        scratch_types=[
            pltpu.VMEM((n_rows, DH), jnp.float32),
            pltpu.VMEM((UNITS * GRP,), jnp.int32),
            pltpu.VMEM((GRP, DH), jnp.float32),
            pltpu.VMEM((GRP, DH), jnp.float32),
            pltpu.SemaphoreType.DMA,
            pltpu.SemaphoreType.DMA,
        ],
    )
    out3 = run(table, idx)
    return out3.transpose(1, 0, 2)


# speedup vs baseline: 1.0302x; 1.0302x over previous
import jax
import jax.numpy as jnp
from jax import lax
from jax.experimental import pallas as pl
from jax.experimental.pallas import tpu as pltpu
from jax.experimental.pallas import tpu_sc as plsc

NC = 2
NS = 16
NW = NC * NS
NP = NW // 2

ROWS_P = 256
GRP = 64
DH = 256
UNITS = 50 * (ROWS_P // GRP)


def _body(table_hbm, idx_hbm, out_hbm, tab_v, idx_v, st0, st1, w0, w1):
    wid = lax.axis_index("s") * NC + lax.axis_index("c")
    pair = wid // 2
    h = lax.rem(wid, 2)
    col0 = pair * ROWS_P
    stage = (st0, st1)
    wsem = (w0, w1)

    pltpu.sync_copy(table_hbm.at[:, pl.ds(h * DH, DH)], tab_v)
    pltpu.sync_copy(idx_hbm.at[pair], idx_v)

    lane = lax.iota(jnp.int32, 16)

    def dst_of(u):
        j = u // 4
        g = lax.rem(u, 4)
        return out_hbm.at[j, pl.ds(col0 + g * GRP, GRP), pl.ds(h * DH, DH)]

    def expand(u, b):
        def row16(r16, carry):
            r0 = r16 * 16
            iv = idx_v[pl.ds(u * GRP + r0, 16)]
            for j in range(16):
                rid = iv[j]
                for k in range(DH // 16):
                    v = tab_v[rid, pl.ds(k * 16, 16)]
                    stage[b][r0 + j, pl.ds(k * 16, 16)] = v
            return carry

        lax.fori_loop(0, GRP // 16, row16, 0)

    def step(gi, carry):
        for b in range(2):
            u = gi * 2 + b

            @pl.when(u >= 2)
            def _():
                pltpu.make_async_copy(stage[b], dst_of(u - 2), wsem[b]).wait()

            expand(u, b)
            pltpu.async_copy(stage[b], dst_of(u), wsem[b])

        return carry

    lax.fori_loop(0, UNITS // 2, step, 0)

    pltpu.make_async_copy(st0, dst_of(UNITS - 2), w0).wait()
    pltpu.make_async_copy(st1, dst_of(UNITS - 1), w1).wait()


def kernel(ids, kernel):
    table = kernel
    n_rows, d = table.shape
    nb_rows, seq = ids.shape
    assert nb_rows == NP * ROWS_P and d == 2 * DH

    idx = (
        ids.astype(jnp.int32)
        .T.reshape(seq, NP, ROWS_P // GRP, GRP)
        .transpose(1, 0, 2, 3)
        .reshape(NP, UNITS * GRP)
    )

    mesh = plsc.VectorSubcoreMesh(
        core_axis_name="c", subcore_axis_name="s", num_cores=NC, num_subcores=NS
    )
    run = pl.kernel(
        _body,
        out_type=jax.ShapeDtypeStruct((seq, nb_rows, d), table.dtype),
        mesh=mesh,
        compiler_params=pltpu.CompilerParams(needs_layout_passes=False),
        scratch_types=[
            pltpu.VMEM((n_rows, DH), jnp.float32),
            pltpu.VMEM((UNITS * GRP,), jnp.int32),
            pltpu.VMEM((GRP, DH), jnp.float32),
            pltpu.VMEM((GRP, DH), jnp.float32),
            pltpu.SemaphoreType.DMA,
            pltpu.SemaphoreType.DMA,
        ],
    )
    out3 = run(table, idx)
    return out3.transpose(1, 0, 2)


# speedup vs baseline: 2.2869x; 2.2197x over previous
import jax
import jax.numpy as jnp
from jax import lax
from jax.experimental import pallas as pl
from jax.experimental.pallas import tpu as pltpu
from jax.experimental.pallas import tpu_sc as plsc

NC = 2
NS = 16
NW = NC * NS

ROWS_W = 128
HALF = 64
UNITS = 50 * (ROWS_W // HALF)


def _body(table_hbm, idx_hbm, out_hbm, idx_v, st0, st1, g0, g1, w0, w1):
    wid = lax.axis_index("s") * NC + lax.axis_index("c")
    col0 = wid * ROWS_W
    stage = (st0, st1)
    gsem = (g0, g1)
    wsem = (w0, w1)

    pltpu.sync_copy(idx_hbm.at[wid], idx_v)

    def dst_of(u):
        j = u // 2
        h = u % 2
        return out_hbm.at[j, pl.ds(col0 + h * HALF, HALF)]

    pltpu.async_copy(table_hbm.at[idx_v.at[0]], st0, g0)
    pltpu.async_copy(table_hbm.at[idx_v.at[1]], st1, g1)

    def step(g, carry):
        for b in range(2):
            u = g * 2 + b
            pltpu.make_async_copy(table_hbm.at[idx_v.at[u]], stage[b], gsem[b]).wait()
            dst = dst_of(u)
            pltpu.async_copy(stage[b], dst, wsem[b])

            @pl.when(u + 2 < UNITS)
            def _():
                pltpu.make_async_copy(stage[b], dst, wsem[b]).wait()
                pltpu.async_copy(table_hbm.at[idx_v.at[u + 2]], stage[b], gsem[b])

        return carry

    lax.fori_loop(0, UNITS // 2, step, 0)

    pltpu.make_async_copy(st0, dst_of(UNITS - 2), w0).wait()
    pltpu.make_async_copy(st1, dst_of(UNITS - 1), w1).wait()


def kernel(ids, kernel):
    table = kernel
    n_rows, d = table.shape
    nb_rows, seq = ids.shape
    assert nb_rows == NW * ROWS_W

    idx = (
        ids.astype(jnp.int32)
        .T.reshape(seq, NW, ROWS_W // HALF, HALF)
        .transpose(1, 0, 2, 3)
        .reshape(NW, UNITS, HALF)
    )
    idx = idx + (jnp.arange(NW, dtype=jnp.int32) * n_rows)[:, None, None]
    table_rep = jnp.tile(table, (NW, 1))

    mesh = plsc.VectorSubcoreMesh(
        core_axis_name="c", subcore_axis_name="s", num_cores=NC, num_subcores=NS
    )
    run = pl.kernel(
        _body,
        out_type=jax.ShapeDtypeStruct((seq, nb_rows, d), table.dtype),
        mesh=mesh,
        scratch_types=[
            pltpu.VMEM((UNITS, HALF), jnp.int32),
            pltpu.VMEM((HALF, d), jnp.float32),
            pltpu.VMEM((HALF, d), jnp.float32),
            pltpu.SemaphoreType.DMA,
            pltpu.SemaphoreType.DMA,
            pltpu.SemaphoreType.DMA,
            pltpu.SemaphoreType.DMA,
        ],
    )
    out3 = run(table_rep, idx)
    return out3.transpose(1, 0, 2)


# speedup vs baseline: 2.3348x; 1.0210x over previous
import jax
import jax.numpy as jnp
from jax import lax
from jax.experimental import pallas as pl
from jax.experimental.pallas import tpu as pltpu
from jax.experimental.pallas import tpu_sc as plsc

NC = 2
NS = 16
NW = NC * NS

ROWS_W = 128
HALF = 64
UNITS = 50 * (ROWS_W // HALF)


def _body(table_hbm, idx_hbm, out_hbm, idx_v, st0, st1, g0, g1, w0, w1):
    wid = lax.axis_index("s") * NC + lax.axis_index("c")
    col0 = wid * ROWS_W
    stage = (st0, st1)
    gsem = (g0, g1)
    wsem = (w0, w1)

    pltpu.sync_copy(idx_hbm.at[wid], idx_v)

    def dst_of(u):
        j = u // 2
        h = u % 2
        return out_hbm.at[j, pl.ds(col0 + h * HALF, HALF)]

    pltpu.async_copy(table_hbm.at[idx_v.at[0]], st0, g0)
    pltpu.async_copy(table_hbm.at[idx_v.at[1]], st1, g1)

    def step(g, carry):
        for b in range(2):
            u = g * 2 + b
            pltpu.make_async_copy(table_hbm.at[idx_v.at[u]], stage[b], gsem[b]).wait()
            dst = dst_of(u)
            pltpu.async_copy(stage[b], dst, wsem[b])

            @pl.when(u + 2 < UNITS)
            def _():
                pltpu.make_async_copy(stage[b], dst, wsem[b]).wait()
                pltpu.async_copy(table_hbm.at[idx_v.at[u + 2]], stage[b], gsem[b])

        return carry

    lax.fori_loop(0, UNITS // 2, step, 0)

    pltpu.make_async_copy(st0, dst_of(UNITS - 2), w0).wait()
    pltpu.make_async_copy(st1, dst_of(UNITS - 1), w1).wait()


def kernel(ids, kernel):
    table = kernel
    n_rows, d = table.shape
    nb_rows, seq = ids.shape
    assert nb_rows == NW * ROWS_W

    idx = (
        ids.astype(jnp.int32)
        .T.reshape(seq, NW, ROWS_W // HALF, HALF)
        .transpose(1, 0, 2, 3)
        .reshape(NW, UNITS, HALF)
    )
    idx = idx + ((jnp.arange(NW, dtype=jnp.int32) // 2) * n_rows)[:, None, None]
    table_rep = jnp.tile(table, (NW // 2, 1))

    mesh = plsc.VectorSubcoreMesh(
        core_axis_name="c", subcore_axis_name="s", num_cores=NC, num_subcores=NS
    )
    run = pl.kernel(
        _body,
        out_type=jax.ShapeDtypeStruct((seq, nb_rows, d), table.dtype),
        mesh=mesh,
        scratch_types=[
            pltpu.VMEM((UNITS, HALF), jnp.int32),
            pltpu.VMEM((HALF, d), jnp.float32),
            pltpu.VMEM((HALF, d), jnp.float32),
            pltpu.SemaphoreType.DMA,
            pltpu.SemaphoreType.DMA,
            pltpu.SemaphoreType.DMA,
            pltpu.SemaphoreType.DMA,
        ],
    )
    out3 = run(table_rep, idx)
    return out3.transpose(1, 0, 2)


# speedup vs baseline: 2.3791x; 1.0190x over previous
import jax
import jax.numpy as jnp
from jax import lax
from jax.experimental import pallas as pl
from jax.experimental.pallas import tpu as pltpu
from jax.experimental.pallas import tpu_sc as plsc

NC = 2
NS = 16
NW = NC * NS

ROWS_W = 128
HALF = 64
UNITS = 50 * (ROWS_W // HALF)


def _body(table_hbm, idx_hbm, out_hbm, idx_v, st0, st1, g0, g1, w0, w1):
    wid = lax.axis_index("s") * NC + lax.axis_index("c")
    col0 = wid * ROWS_W
    stage = (st0, st1)
    gsem = (g0, g1)
    wsem = (w0, w1)

    pltpu.sync_copy(idx_hbm.at[wid], idx_v)

    def dst_of(u):
        j = u // 2
        h = u % 2
        return out_hbm.at[j, pl.ds(col0 + h * HALF, HALF)]

    pltpu.async_copy(table_hbm.at[idx_v.at[0]], st0, g0)
    pltpu.async_copy(table_hbm.at[idx_v.at[1]], st1, g1)

    def step(g, carry):
        for b in range(2):
            u = g * 2 + b
            pltpu.make_async_copy(table_hbm.at[idx_v.at[u]], stage[b], gsem[b]).wait()
            dst = dst_of(u)
            pltpu.async_copy(stage[b], dst, wsem[b])

            @pl.when(u + 2 < UNITS)
            def _():
                pltpu.make_async_copy(stage[b], dst, wsem[b]).wait()
                pltpu.async_copy(table_hbm.at[idx_v.at[u + 2]], stage[b], gsem[b])

        return carry

    lax.fori_loop(0, UNITS // 2, step, 0)

    pltpu.make_async_copy(st0, dst_of(UNITS - 2), w0).wait()
    pltpu.make_async_copy(st1, dst_of(UNITS - 1), w1).wait()


def kernel(ids, kernel):
    table = kernel
    n_rows, d = table.shape
    nb_rows, seq = ids.shape
    assert nb_rows == NW * ROWS_W

    idx = (
        ids.astype(jnp.int32)
        .T.reshape(seq, NW, ROWS_W // HALF, HALF)
        .transpose(1, 0, 2, 3)
        .reshape(NW, UNITS, HALF)
    )
    idx = idx + ((jnp.arange(NW, dtype=jnp.int32) // 4) * n_rows)[:, None, None]
    table_rep = jnp.tile(table, (NW // 4, 1))

    mesh = plsc.VectorSubcoreMesh(
        core_axis_name="c", subcore_axis_name="s", num_cores=NC, num_subcores=NS
    )
    run = pl.kernel(
        _body,
        out_type=jax.ShapeDtypeStruct((seq, nb_rows, d), table.dtype),
        mesh=mesh,
        scratch_types=[
            pltpu.VMEM((UNITS, HALF), jnp.int32),
            pltpu.VMEM((HALF, d), jnp.float32),
            pltpu.VMEM((HALF, d), jnp.float32),
            pltpu.SemaphoreType.DMA,
            pltpu.SemaphoreType.DMA,
            pltpu.SemaphoreType.DMA,
            pltpu.SemaphoreType.DMA,
        ],
    )
    out3 = run(table_rep, idx)
    return out3.transpose(1, 0, 2)
